# baseline (speedup 1.0000x reference)
import jax
import jax.numpy as jnp
from jax import lax
from jax.experimental import pallas as pl
from jax.experimental.pallas import tpu as pltpu
from jax.experimental.pallas import tpu_sc as plsc

NC, NS = 2, 16
NW = NC * NS
STR = 80
CH = 400

N = 10000
E = 320000
D = 128
DE = 16
H = 128
DI = 16

HE = E // 2
EPW = HE // NW
ZR = 1000
NZW = N // ZR
BE = 3200
WIDE = N * DI // 128


def _chunk_plan(epw):
    plan = []
    off = 0
    while off < epw:
        n = min(CH, epw - off)
        streams = []
        r = n
        while r:
            ln = min(STR, r)
            streams.append(ln)
            r -= ln
        plan.append((off, streams))
        off += n
    return plan


def _xa_body(x_ref, w_ref, b_ref, o_ref):
    o_ref[...] = (
        jnp.dot(x_ref[...], w_ref[...], preferred_element_type=jnp.float32)
        + b_ref[...]
    ).astype(jnp.bfloat16)


def _mlp_body(g_ref, eat_ref, w1b_ref, w2p_ref, b2p_ref, o_ref):
    eb = lax.dot_general(
        eat_ref[...], w1b_ref[...], (((0,), (0,)), ((), ())),
        preferred_element_type=jnp.float32,
    )
    h = jnp.maximum(g_ref[...].astype(jnp.float32) + eb, 0.0)
    o_ref[...] = (
        jnp.dot(h, w2p_ref[...], preferred_element_type=jnp.float32) + b2p_ref[...]
    )


def _div_body(s1_ref, c1_ref, s2_ref, c2_ref, o_ref):
    sm = s1_ref[0] + s1_ref[1] + s2_ref[0] + s2_ref[1]
    ct = c1_ref[0] + c1_ref[1] + c2_ref[0] + c2_ref[1]
    o_ref[...] = sm / jnp.maximum(ct, 1.0)


_sc_mesh = plsc.VectorSubcoreMesh(
    core_axis_name="c", subcore_axis_name="s", num_cores=NC, num_subcores=NS
)
_sc_params = pltpu.CompilerParams(use_tc_tiling_on_sc=False)


def _make_gather(off):
    plan = _chunk_plan(EPW)

    def body(xa_hbm, idx_hbm, g_hbm, idx_v, rows_v, gsem, wsem):
        c = lax.axis_index("c")
        s = lax.axis_index("s")
        base = off + (c * NS + s) * EPW

        wb = [None, None]
        for t, (co, streams) in enumerate(plan):
            b = t % 2
            if t >= 2:
                wb[b].wait()
            e0 = base + co
            n = sum(streams)
            pltpu.sync_copy(idx_hbm.at[pl.ds(e0, n)], idx_v.at[b].at[pl.ds(0, n)])
            fired = []
            so = 0
            for ln in streams:
                fired.append(pltpu.async_copy(
                    xa_hbm.at[idx_v.at[b].at[pl.ds(so, ln)]],
                    rows_v.at[b].at[pl.ds(so, ln)],
                    gsem.at[b],
                ))
                so += ln
            for f in fired:
                f.wait()
            wb[b] = pltpu.async_copy(
                rows_v.at[b].at[pl.ds(0, n)],
                g_hbm.at[pl.ds(e0 - off, n)], wsem.at[b])
        for b in {(len(plan) - 2) % 2, (len(plan) - 1) % 2}:
            wb[b].wait()

    return pl.kernel(
        body,
        compiler_params=_sc_params,
        out_type=jax.ShapeDtypeStruct((HE, H), jnp.bfloat16),
        mesh=_sc_mesh,
        scratch_types=[
            pltpu.VMEM((2, CH), jnp.int32),
            pltpu.VMEM((2, CH, H), jnp.bfloat16),
            pltpu.SemaphoreType.DMA((2,)),
            pltpu.SemaphoreType.DMA((2,)),
        ],
    )


def _repack_out(acc, t1, t2, out_hbm, core, part):
    ext = 290 if part == 3 else 320
    for half in range(2):
        hw = min(ext - 160 * half, 160)
        w0 = part * 320 + 160 * half
        pltpu.sync_copy(acc.at[pl.ds(w0 * 8, hw * 8)], t1.at[pl.ds(0, hw * 8)])

        def rows(q, carry):
            for j in range(8):
                t2[q, pl.ds(j * 16, 16)] = t1[q * 8 + j]
            return carry

        lax.fori_loop(0, hw, rows, 0)
        pltpu.sync_copy(t2.at[pl.ds(0, hw)], out_hbm.at[core, pl.ds(w0, hw)])


def _make_scatter(off):
    plan = _chunk_plan(EPW)

    def body(col_hbm, en_hbm, sums_hbm, cnts_hbm,
             cidx_v, dat_v, one_v, zer_v, t1_v, t2_v, acc_s, acc_c, ssem, asem):
        c = lax.axis_index("c")
        s = lax.axis_index("s")
        base = off + (c * NS + s) * EPW

        def fill_z(i, carry):
            zer_v[i] = jnp.zeros((16,), jnp.float32)
            return carry

        lax.fori_loop(0, ZR, fill_z, 0)

        def fill_o(i, carry):
            one_v[i] = jnp.full((16,), 1.0, jnp.float32)
            return carry

        lax.fori_loop(0, STR, fill_o, 0)

        @pl.when(s < NZW)
        def _():
            pltpu.sync_copy(zer_v, acc_s.at[pl.ds(s * ZR, ZR)])
            pltpu.sync_copy(zer_v, acc_c.at[pl.ds(s * ZR, ZR)])

        plsc.subcore_barrier()

        adds = [None, None]
        for t, (co, streams) in enumerate(plan):
            b = t % 2
            if t >= 2:
                for f in adds[b]:
                    f.wait()
            e0 = base + co
            n = sum(streams)
            st1 = pltpu.async_copy(col_hbm.at[pl.ds(e0, n)],
                                   cidx_v.at[b].at[pl.ds(0, n)], ssem.at[b])
            st2 = pltpu.async_copy(en_hbm.at[pl.ds(e0 - off, n), pl.ds(0, DI)],
                                   dat_v.at[b].at[pl.ds(0, n)], ssem.at[b])
            st1.wait()
            st2.wait()
            fired = []
            so = 0
            for ln in streams:
                ck = cidx_v.at[b].at[pl.ds(so, ln)]
                fired.append(pltpu.async_copy(
                    dat_v.at[b].at[pl.ds(so, ln)],
                    acc_s.at[ck], asem.at[b], add=True))
                fired.append(pltpu.async_copy(
                    one_v.at[pl.ds(0, ln)], acc_c.at[ck], asem.at[b], add=True))
                so += ln
            adds[b] = fired
        for b in {(len(plan) - 2) % 2, (len(plan) - 1) % 2}:
            for f in adds[b]:
                f.wait()
        plsc.subcore_barrier()

        for part in range(4):
            @pl.when(s == part)
            def _(part=part):
                _repack_out(acc_s, t1_v, t2_v, sums_hbm, c, part)

            @pl.when(s == 4 + part)
            def _(part=part):
                _repack_out(acc_c, t1_v, t2_v, cnts_hbm, c, part)

    return pl.kernel(
        body,
        compiler_params=_sc_params,
        out_type=(
            jax.ShapeDtypeStruct((NC, WIDE, 128), jnp.float32),
            jax.ShapeDtypeStruct((NC, WIDE, 128), jnp.float32),
        ),
        mesh=_sc_mesh,
        scratch_types=[
            pltpu.VMEM((2, CH), jnp.int32),
            pltpu.VMEM((2, CH, DI), jnp.float32),
            pltpu.VMEM((STR, DI), jnp.float32),
            pltpu.VMEM((ZR, DI), jnp.float32),
            pltpu.VMEM((1280, DI), jnp.float32),
            pltpu.VMEM((160, 128), jnp.float32),
            pltpu.VMEM_SHARED((N, DI), jnp.float32),
            pltpu.VMEM_SHARED((N, DI), jnp.float32),
            pltpu.SemaphoreType.DMA((2,)),
            pltpu.SemaphoreType.DMA((2,)),
        ],
    )


_gather1 = _make_gather(0)
_gather2 = _make_gather(HE)
_scatter1 = _make_scatter(0)
_scatter2 = _make_scatter(HE)


def _make_mlp(off_blocks):
    return pl.pallas_call(
        _mlp_body,
        grid=(HE // BE,),
        in_specs=[
            pl.BlockSpec((BE, H), lambda i: (i, 0)),
            pl.BlockSpec((DE, BE), lambda i: (0, i + off_blocks)),
            pl.BlockSpec((DE, H), lambda i: (0, 0)),
            pl.BlockSpec((H, H), lambda i: (0, 0)),
            pl.BlockSpec((1, H), lambda i: (0, 0)),
        ],
        out_specs=pl.BlockSpec((BE, H), lambda i: (i, 0)),
        out_shape=jax.ShapeDtypeStruct((HE, H), jnp.float32),
    )


_mlp1 = _make_mlp(0)
_mlp2 = _make_mlp(HE // BE)

_xa = pl.pallas_call(
    _xa_body,
    out_shape=jax.ShapeDtypeStruct((N, H), jnp.bfloat16),
)

_div = pl.pallas_call(
    _div_body,
    out_shape=jax.ShapeDtypeStruct((WIDE, 128), jnp.float32),
)


def kernel(x, edge_index, edge_attr, W1, b1, W2, b2):
    row = edge_index[0]
    col = edge_index[1]

    xa = _xa(x, W1[:D], b1.reshape(1, H))
    eat = edge_attr.T
    w2p = jnp.concatenate([W2, jnp.zeros((H, H - DI), jnp.float32)], axis=1)
    b2p = jnp.concatenate([b2, jnp.zeros((H - DI,), jnp.float32)]).reshape(1, H)

    g1 = _gather1(xa, row)
    g2 = _gather2(xa, row)
    en1 = _mlp1(g1, eat, W1[D:], w2p, b2p)
    en2 = _mlp2(g2, eat, W1[D:], w2p, b2p)
    s1, c1 = _scatter1(col, en1)
    s2, c2 = _scatter2(col, en2)
    node_wide = _div(s1, c1, s2, c2)
    node_new = node_wide.reshape(N, DI)
    edge_new = jnp.concatenate([en1[:, :DI], en2[:, :DI]], axis=0)
    return (node_new, edge_new)


# speedup vs baseline: 1.6913x; 1.6913x over previous
import jax
import jax.numpy as jnp
from jax import lax
from jax.experimental import pallas as pl
from jax.experimental.pallas import tpu as pltpu
from jax.experimental.pallas import tpu_sc as plsc

NC, NS = 2, 16
NW = NC * NS
STR = 80
CH = 400

N = 10000
E = 320000
D = 128
DE = 16
H = 128
DI = 16

HE = E // 2
EPW = HE // NW
ZR = 1000
NZW = N // ZR
BE = 3200
WIDE = N * DI // 128


def _chunk_plan(epw):
    plan = []
    off = 0
    while off < epw:
        n = min(CH, epw - off)
        streams = []
        r = n
        while r:
            ln = min(STR, r)
            streams.append(ln)
            r -= ln
        plan.append((off, streams))
        off += n
    return plan


def _xa_body(x_ref, w_ref, b_ref, o_ref):
    o_ref[...] = (
        jnp.dot(x_ref[...], w_ref[...], preferred_element_type=jnp.float32)
        + b_ref[...]
    )


def _mlp_body(g_ref, eat_ref, w1b_ref, w2p_ref, b2p_ref, o_ref):
    eb = lax.dot_general(
        eat_ref[...], w1b_ref[...], (((0,), (0,)), ((), ())),
        preferred_element_type=jnp.float32,
    )
    h = jnp.maximum(g_ref[...] + eb, 0.0)
    o_ref[...] = (
        jnp.dot(h, w2p_ref[...], preferred_element_type=jnp.float32) + b2p_ref[...]
    )


def _div_body(s1_ref, c1_ref, s2_ref, c2_ref, o_ref):
    sm = s1_ref[0] + s1_ref[1] + s2_ref[0] + s2_ref[1]
    ct = c1_ref[0] + c1_ref[1] + c2_ref[0] + c2_ref[1]
    o_ref[...] = sm / jnp.maximum(ct, 1.0)


_sc_mesh = plsc.VectorSubcoreMesh(
    core_axis_name="c", subcore_axis_name="s", num_cores=NC, num_subcores=NS
)
_sc_params = pltpu.CompilerParams(use_tc_tiling_on_sc=False)


def _make_gather(off):
    plan = _chunk_plan(EPW)

    def body(xa_hbm, idx_hbm, g_hbm, idx_v, rows_v, gsem, wsem):
        c = lax.axis_index("c")
        s = lax.axis_index("s")
        base = off + (c * NS + s) * EPW

        wb = [None, None]
        for t, (co, streams) in enumerate(plan):
            b = t % 2
            if t >= 2:
                wb[b].wait()
            e0 = base + co
            n = sum(streams)
            pltpu.sync_copy(idx_hbm.at[pl.ds(e0, n)], idx_v.at[b].at[pl.ds(0, n)])
            fired = []
            so = 0
            for ln in streams:
                fired.append(pltpu.async_copy(
                    xa_hbm.at[idx_v.at[b].at[pl.ds(so, ln)]],
                    rows_v.at[b].at[pl.ds(so, ln)],
                    gsem.at[b],
                ))
                so += ln
            for f in fired:
                f.wait()
            wb[b] = pltpu.async_copy(
                rows_v.at[b].at[pl.ds(0, n)],
                g_hbm.at[pl.ds(e0 - off, n)], wsem.at[b])
        for b in {(len(plan) - 2) % 2, (len(plan) - 1) % 2}:
            wb[b].wait()

    return pl.kernel(
        body,
        compiler_params=_sc_params,
        out_type=jax.ShapeDtypeStruct((HE, H), jnp.float32),
        mesh=_sc_mesh,
        scratch_types=[
            pltpu.VMEM((2, CH), jnp.int32),
            pltpu.VMEM((2, CH, H), jnp.float32),
            pltpu.SemaphoreType.DMA((2,)),
            pltpu.SemaphoreType.DMA((2,)),
        ],
    )


def _repack_out(acc, t1, t2, out_hbm, core, part):
    ext = 290 if part == 3 else 320
    for half in range(2):
        hw = min(ext - 160 * half, 160)
        w0 = part * 320 + 160 * half
        pltpu.sync_copy(acc.at[pl.ds(w0 * 8, hw * 8)], t1.at[pl.ds(0, hw * 8)])

        def rows(q, carry):
            for j in range(8):
                t2[q, pl.ds(j * 16, 16)] = t1[q * 8 + j]
            return carry

        lax.fori_loop(0, hw, rows, 0)
        pltpu.sync_copy(t2.at[pl.ds(0, hw)], out_hbm.at[core, pl.ds(w0, hw)])


def _make_scatter(off):
    plan = _chunk_plan(EPW)

    def body(col_hbm, en_hbm, sums_hbm, cnts_hbm,
             cidx_v, dat_v, one_v, zer_v, t1_v, t2_v, acc_s, acc_c, ssem, asem):
        c = lax.axis_index("c")
        s = lax.axis_index("s")
        base = off + (c * NS + s) * EPW

        def fill_z(i, carry):
            zer_v[i] = jnp.zeros((16,), jnp.float32)
            return carry

        lax.fori_loop(0, ZR, fill_z, 0)

        def fill_o(i, carry):
            one_v[i] = jnp.full((16,), 1.0, jnp.float32)
            return carry

        lax.fori_loop(0, STR, fill_o, 0)

        @pl.when(s < NZW)
        def _():
            pltpu.sync_copy(zer_v, acc_s.at[pl.ds(s * ZR, ZR)])
            pltpu.sync_copy(zer_v, acc_c.at[pl.ds(s * ZR, ZR)])

        plsc.subcore_barrier()

        adds = [None, None]
        for t, (co, streams) in enumerate(plan):
            b = t % 2
            if t >= 2:
                for f in adds[b]:
                    f.wait()
            e0 = base + co
            n = sum(streams)
            st1 = pltpu.async_copy(col_hbm.at[pl.ds(e0, n)],
                                   cidx_v.at[b].at[pl.ds(0, n)], ssem.at[b])
            st2 = pltpu.async_copy(en_hbm.at[pl.ds(e0 - off, n), pl.ds(0, DI)],
                                   dat_v.at[b].at[pl.ds(0, n)], ssem.at[b])
            st1.wait()
            st2.wait()
            fired = []
            so = 0
            for ln in streams:
                ck = cidx_v.at[b].at[pl.ds(so, ln)]
                fired.append(pltpu.async_copy(
                    dat_v.at[b].at[pl.ds(so, ln)],
                    acc_s.at[ck], asem.at[b], add=True))
                fired.append(pltpu.async_copy(
                    one_v.at[pl.ds(0, ln)], acc_c.at[ck], asem.at[b], add=True))
                so += ln
            adds[b] = fired
        for b in {(len(plan) - 2) % 2, (len(plan) - 1) % 2}:
            for f in adds[b]:
                f.wait()
        plsc.subcore_barrier()

        for part in range(4):
            @pl.when(s == part)
            def _(part=part):
                _repack_out(acc_s, t1_v, t2_v, sums_hbm, c, part)

            @pl.when(s == 4 + part)
            def _(part=part):
                _repack_out(acc_c, t1_v, t2_v, cnts_hbm, c, part)

    return pl.kernel(
        body,
        compiler_params=_sc_params,
        out_type=(
            jax.ShapeDtypeStruct((NC, WIDE, 128), jnp.float32),
            jax.ShapeDtypeStruct((NC, WIDE, 128), jnp.float32),
        ),
        mesh=_sc_mesh,
        scratch_types=[
            pltpu.VMEM((2, CH), jnp.int32),
            pltpu.VMEM((2, CH, DI), jnp.float32),
            pltpu.VMEM((STR, DI), jnp.float32),
            pltpu.VMEM((ZR, DI), jnp.float32),
            pltpu.VMEM((1280, DI), jnp.float32),
            pltpu.VMEM((160, 128), jnp.float32),
            pltpu.VMEM_SHARED((N, DI), jnp.float32),
            pltpu.VMEM_SHARED((N, DI), jnp.float32),
            pltpu.SemaphoreType.DMA((2,)),
            pltpu.SemaphoreType.DMA((2,)),
        ],
    )


_gather1 = _make_gather(0)
_gather2 = _make_gather(HE)
_scatter1 = _make_scatter(0)
_scatter2 = _make_scatter(HE)


def _make_mlp(off_blocks):
    return pl.pallas_call(
        _mlp_body,
        grid=(HE // BE,),
        in_specs=[
            pl.BlockSpec((BE, H), lambda i: (i, 0)),
            pl.BlockSpec((DE, BE), lambda i: (0, i + off_blocks)),
            pl.BlockSpec((DE, H), lambda i: (0, 0)),
            pl.BlockSpec((H, H), lambda i: (0, 0)),
            pl.BlockSpec((1, H), lambda i: (0, 0)),
        ],
        out_specs=pl.BlockSpec((BE, H), lambda i: (i, 0)),
        out_shape=jax.ShapeDtypeStruct((HE, H), jnp.float32),
    )


_mlp1 = _make_mlp(0)
_mlp2 = _make_mlp(HE // BE)

_xa = pl.pallas_call(
    _xa_body,
    out_shape=jax.ShapeDtypeStruct((N, H), jnp.float32),
)

_div = pl.pallas_call(
    _div_body,
    out_shape=jax.ShapeDtypeStruct((WIDE, 128), jnp.float32),
)


def kernel(x, edge_index, edge_attr, W1, b1, W2, b2):
    row = edge_index[0]
    col = edge_index[1]

    xa = _xa(x, W1[:D], b1.reshape(1, H))
    eat = edge_attr.T
    w2p = jnp.concatenate([W2, jnp.zeros((H, H - DI), jnp.float32)], axis=1)
    b2p = jnp.concatenate([b2, jnp.zeros((H - DI,), jnp.float32)]).reshape(1, H)

    g1 = _gather1(xa, row)
    g2 = _gather2(xa, row)
    en1 = _mlp1(g1, eat, W1[D:], w2p, b2p)
    en2 = _mlp2(g2, eat, W1[D:], w2p, b2p)
    s1, c1 = _scatter1(col, en1)
    s2, c2 = _scatter2(col, en2)
    node_wide = _div(s1, c1, s2, c2)
    node_new = node_wide.reshape(N, DI)
    edge_new = jnp.concatenate([en1[:, :DI], en2[:, :DI]], axis=0)
    return (node_new, edge_new)


# speedup vs baseline: 1.7718x; 1.0476x over previous
import jax
import jax.numpy as jnp
from jax import lax
from jax.experimental import pallas as pl
from jax.experimental.pallas import tpu as pltpu
from jax.experimental.pallas import tpu_sc as plsc

NC, NS = 2, 16
NW = NC * NS
STR = 80
CH = 400

N = 10000
E = 320000
D = 128
DE = 16
H = 128
DI = 16

HE = E // 2
EPW = HE // NW
ZR = 1000
NZW = N // ZR
BE = 3200
WIDE = N * DI // 128


def _chunk_plan(epw):
    plan = []
    off = 0
    while off < epw:
        n = min(CH, epw - off)
        streams = []
        r = n
        while r:
            ln = min(STR, r)
            streams.append(ln)
            r -= ln
        plan.append((off, streams))
        off += n
    return plan


def _xa_body(x_ref, w_ref, b_ref, o_ref):
    o_ref[...] = (
        jnp.dot(x_ref[...], w_ref[...], preferred_element_type=jnp.float32)
        + b_ref[...]
    )


def _mlp_body(g_ref, eat_ref, w1b_ref, w2p_ref, b2p_ref, o_ref):
    eb = lax.dot_general(
        eat_ref[...], w1b_ref[...], (((0,), (0,)), ((), ())),
        preferred_element_type=jnp.float32,
    )
    h = jnp.maximum(g_ref[...] + eb, 0.0)
    o_ref[...] = (
        jnp.dot(h, w2p_ref[...], preferred_element_type=jnp.float32) + b2p_ref[...]
    )


def _div_body(s_ref, c_ref, o_ref):
    sm = s_ref[0] + s_ref[1]
    ct = c_ref[0] + c_ref[1]
    o_ref[...] = sm / jnp.maximum(ct, 1.0)


_sc_mesh = plsc.VectorSubcoreMesh(
    core_axis_name="c", subcore_axis_name="s", num_cores=NC, num_subcores=NS
)
_sc_params = pltpu.CompilerParams(use_tc_tiling_on_sc=False)


def _make_gather(off):
    plan = _chunk_plan(EPW)

    def body(xa_hbm, idx_hbm, g_hbm, idx_v, rows_v, gsem, wsem):
        c = lax.axis_index("c")
        s = lax.axis_index("s")
        base = off + (c * NS + s) * EPW

        wb = [None, None]
        for t, (co, streams) in enumerate(plan):
            b = t % 2
            if t >= 2:
                wb[b].wait()
            e0 = base + co
            n = sum(streams)
            pltpu.sync_copy(idx_hbm.at[pl.ds(e0, n)], idx_v.at[b].at[pl.ds(0, n)])
            fired = []
            so = 0
            for ln in streams:
                fired.append(pltpu.async_copy(
                    xa_hbm.at[idx_v.at[b].at[pl.ds(so, ln)]],
                    rows_v.at[b].at[pl.ds(so, ln)],
                    gsem.at[b],
                ))
                so += ln
            for f in fired:
                f.wait()
            wb[b] = pltpu.async_copy(
                rows_v.at[b].at[pl.ds(0, n)],
                g_hbm.at[pl.ds(e0 - off, n)], wsem.at[b])
        for b in {(len(plan) - 2) % 2, (len(plan) - 1) % 2}:
            wb[b].wait()

    return pl.kernel(
        body,
        compiler_params=_sc_params,
        out_type=jax.ShapeDtypeStruct((HE, H), jnp.float32),
        mesh=_sc_mesh,
        scratch_types=[
            pltpu.VMEM((2, CH), jnp.int32),
            pltpu.VMEM((2, CH, H), jnp.float32),
            pltpu.SemaphoreType.DMA((2,)),
            pltpu.SemaphoreType.DMA((2,)),
        ],
    )


def _repack_out(acc, t1, t2, out_hbm, core, part):
    ext = 290 if part == 3 else 320
    for half in range(2):
        hw = min(ext - 160 * half, 160)
        w0 = part * 320 + 160 * half
        pltpu.sync_copy(acc.at[pl.ds(w0 * 8, hw * 8)], t1.at[pl.ds(0, hw * 8)])

        def rows(q, carry):
            for j in range(8):
                t2[q, pl.ds(j * 16, 16)] = t1[q * 8 + j]
            return carry

        lax.fori_loop(0, hw, rows, 0)
        pltpu.sync_copy(t2.at[pl.ds(0, hw)], out_hbm.at[core, pl.ds(w0, hw)])


def _make_scatter():
    epw = HE // NS
    plan = _chunk_plan(epw)

    def _half(col_hbm, en_hbm, acc_s, acc_c,
              cidx_v, dat_v, one_v, ssem, asem, s, half):
        base_col = half * HE + s * epw
        base_en = s * epw

        adds = [None, None]
        for t, (co, streams) in enumerate(plan):
            b = t % 2
            if t >= 2:
                for f in adds[b]:
                    f.wait()
            n = sum(streams)
            st1 = pltpu.async_copy(col_hbm.at[pl.ds(base_col + co, n)],
                                   cidx_v.at[b].at[pl.ds(0, n)], ssem.at[b])
            st2 = pltpu.async_copy(en_hbm.at[pl.ds(base_en + co, n), pl.ds(0, DI)],
                                   dat_v.at[b].at[pl.ds(0, n)], ssem.at[b])
            st1.wait()
            st2.wait()
            fired = []
            so = 0
            for ln in streams:
                ck = cidx_v.at[b].at[pl.ds(so, ln)]
                fired.append(pltpu.async_copy(
                    dat_v.at[b].at[pl.ds(so, ln)],
                    acc_s.at[ck], asem.at[b], add=True))
                fired.append(pltpu.async_copy(
                    one_v.at[pl.ds(0, ln)], acc_c.at[ck], asem.at[b], add=True))
                so += ln
            adds[b] = fired
        for b in {(len(plan) - 2) % 2, (len(plan) - 1) % 2}:
            for f in adds[b]:
                f.wait()

    def body(col_hbm, en1_hbm, en2_hbm, sums_hbm, cnts_hbm,
             cidx_v, dat_v, one_v, zer_v, t1_v, t2_v, acc_s, acc_c, ssem, asem):
        c = lax.axis_index("c")
        s = lax.axis_index("s")

        def fill_z(i, carry):
            zer_v[i] = jnp.zeros((16,), jnp.float32)
            return carry

        lax.fori_loop(0, ZR, fill_z, 0)

        def fill_o(i, carry):
            one_v[i] = jnp.full((16,), 1.0, jnp.float32)
            return carry

        lax.fori_loop(0, STR, fill_o, 0)

        @pl.when(s < NZW)
        def _():
            pltpu.sync_copy(zer_v, acc_s.at[pl.ds(s * ZR, ZR)])
            pltpu.sync_copy(zer_v, acc_c.at[pl.ds(s * ZR, ZR)])

        plsc.subcore_barrier()

        @pl.when(c == 0)
        def _():
            _half(col_hbm, en1_hbm, acc_s, acc_c,
                  cidx_v, dat_v, one_v, ssem, asem, s, 0)

        @pl.when(c == 1)
        def _():
            _half(col_hbm, en2_hbm, acc_s, acc_c,
                  cidx_v, dat_v, one_v, ssem, asem, s, 1)

        plsc.subcore_barrier()

        for part in range(4):
            @pl.when(s == part)
            def _(part=part):
                _repack_out(acc_s, t1_v, t2_v, sums_hbm, c, part)

            @pl.when(s == 4 + part)
            def _(part=part):
                _repack_out(acc_c, t1_v, t2_v, cnts_hbm, c, part)

    return pl.kernel(
        body,
        compiler_params=_sc_params,
        out_type=(
            jax.ShapeDtypeStruct((NC, WIDE, 128), jnp.float32),
            jax.ShapeDtypeStruct((NC, WIDE, 128), jnp.float32),
        ),
        mesh=_sc_mesh,
        scratch_types=[
            pltpu.VMEM((2, CH), jnp.int32),
            pltpu.VMEM((2, CH, DI), jnp.float32),
            pltpu.VMEM((STR, DI), jnp.float32),
            pltpu.VMEM((ZR, DI), jnp.float32),
            pltpu.VMEM((1280, DI), jnp.float32),
            pltpu.VMEM((160, 128), jnp.float32),
            pltpu.VMEM_SHARED((N, DI), jnp.float32),
            pltpu.VMEM_SHARED((N, DI), jnp.float32),
            pltpu.SemaphoreType.DMA((2,)),
            pltpu.SemaphoreType.DMA((2,)),
        ],
    )


_gather1 = _make_gather(0)
_gather2 = _make_gather(HE)
_scatter = _make_scatter()


def _make_mlp(off_blocks):
    return pl.pallas_call(
        _mlp_body,
        grid=(HE // BE,),
        in_specs=[
            pl.BlockSpec((BE, H), lambda i: (i, 0)),
            pl.BlockSpec((DE, BE), lambda i: (0, i + off_blocks)),
            pl.BlockSpec((DE, H), lambda i: (0, 0)),
            pl.BlockSpec((H, H), lambda i: (0, 0)),
            pl.BlockSpec((1, H), lambda i: (0, 0)),
        ],
        out_specs=pl.BlockSpec((BE, H), lambda i: (i, 0)),
        out_shape=jax.ShapeDtypeStruct((HE, H), jnp.float32),
    )


_mlp1 = _make_mlp(0)
_mlp2 = _make_mlp(HE // BE)

_xa = pl.pallas_call(
    _xa_body,
    out_shape=jax.ShapeDtypeStruct((N, H), jnp.float32),
)

_div = pl.pallas_call(
    _div_body,
    out_shape=jax.ShapeDtypeStruct((WIDE, 128), jnp.float32),
)


def kernel(x, edge_index, edge_attr, W1, b1, W2, b2):
    row = edge_index[0]
    col = edge_index[1]

    xa = _xa(x, W1[:D], b1.reshape(1, H))
    eat = edge_attr.T
    w2p = jnp.concatenate([W2, jnp.zeros((H, H - DI), jnp.float32)], axis=1)
    b2p = jnp.concatenate([b2, jnp.zeros((H - DI,), jnp.float32)]).reshape(1, H)

    g1 = _gather1(xa, row)
    g2 = _gather2(xa, row)
    en1 = _mlp1(g1, eat, W1[D:], w2p, b2p)
    en2 = _mlp2(g2, eat, W1[D:], w2p, b2p)
    sums, cnts = _scatter(col, en1, en2)
    node_wide = _div(sums, cnts)
    node_new = node_wide.reshape(N, DI)
    edge_new = jnp.concatenate([en1[:, :DI], en2[:, :DI]], axis=0)
    return (node_new, edge_new)
